# baseline (speedup 1.0000x reference)
import functools

import jax
import jax.numpy as jnp
from jax import lax
from jax.experimental import pallas as pl
from jax.experimental.pallas import tpu as pltpu
from jax.experimental.pallas import tpu_sc as plsc

B = 16384
V = 1000000
D = 64

_NC = 2
_NS = 16
_NW = _NC * _NS
_BPW = B // _NW


def _sc_gather_body(uid_hbm, iid_hbm, ug_hbm, ig_hbm, um_hbm, im_hbm,
                    o_ug, o_ig, o_um, o_im,
                    idx_u, idx_i, buf_a, buf_b, sem_a, sem_b):
    wid = lax.axis_index("s") * _NC + lax.axis_index("c")
    base = wid * _BPW
    pltpu.sync_copy(uid_hbm.at[pl.ds(base, _BPW)], idx_u)
    pltpu.sync_copy(iid_hbm.at[pl.ds(base, _BPW)], idx_i)

    cap_a = pltpu.async_copy(ug_hbm.at[idx_u], buf_a, sem_a)
    cap_b = pltpu.async_copy(ig_hbm.at[idx_i], buf_b, sem_b)
    cap_a.wait()
    pltpu.sync_copy(buf_a, o_ug.at[pl.ds(base, _BPW)])
    cap_a2 = pltpu.async_copy(um_hbm.at[idx_u], buf_a, sem_a)
    cap_b.wait()
    pltpu.sync_copy(buf_b, o_ig.at[pl.ds(base, _BPW)])
    cap_b2 = pltpu.async_copy(im_hbm.at[idx_i], buf_b, sem_b)
    cap_a2.wait()
    pltpu.sync_copy(buf_a, o_um.at[pl.ds(base, _BPW)])
    cap_b2.wait()
    pltpu.sync_copy(buf_b, o_im.at[pl.ds(base, _BPW)])


_sc_gather = functools.partial(
    pl.kernel,
    out_type=[jax.ShapeDtypeStruct((B, D), jnp.float32)] * 4,
    mesh=plsc.VectorSubcoreMesh(core_axis_name="c", subcore_axis_name="s"),
    compiler_params=pltpu.CompilerParams(use_tc_tiling_on_sc=False),
    scratch_types=[
        pltpu.VMEM((_BPW,), jnp.int32),
        pltpu.VMEM((_BPW,), jnp.int32),
        pltpu.VMEM((_BPW, D), jnp.float32),
        pltpu.VMEM((_BPW, D), jnp.float32),
        pltpu.SemaphoreType.DMA,
        pltpu.SemaphoreType.DMA,
    ],
)(_sc_gather_body)


_BLK = 2048


def _tc_body(ug, ig, um, im, w1a, w1b, b1, w2, b2, w3, b3, wog, woh, bo, out):
    f32 = jnp.float32
    gmf = ug[...] * ig[...]
    h = jnp.dot(um[...], w1a[...], preferred_element_type=f32)
    h += jnp.dot(im[...], w1b[...], preferred_element_type=f32)
    h = jnp.maximum(h + b1[...], 0.0)
    h = jnp.maximum(jnp.dot(h, w2[...], preferred_element_type=f32) + b2[...], 0.0)
    h = jnp.maximum(jnp.dot(h, w3[...], preferred_element_type=f32) + b3[...], 0.0)
    logit = jnp.sum(gmf * wog[...], axis=1) + jnp.sum(h * woh[...], axis=1)
    out[...] = jax.nn.sigmoid(logit + bo[0, 0])


def _tc_dense(ug, ig, um, im, w1a, w1b, b1, w2, b2, w3, b3, wog, woh, bo):
    n_blk = B // _BLK
    row_spec = pl.BlockSpec((_BLK, D), lambda i: (i, 0))
    full = lambda a: pl.BlockSpec(a.shape, lambda i: (0,) * a.ndim)
    return pl.pallas_call(
        _tc_body,
        grid=(n_blk,),
        in_specs=[row_spec, row_spec, row_spec, row_spec,
                  full(w1a), full(w1b), full(b1), full(w2), full(b2),
                  full(w3), full(b3), full(wog), full(woh), full(bo)],
        out_specs=pl.BlockSpec((_BLK,), lambda i: (i,)),
        out_shape=jax.ShapeDtypeStruct((B,), jnp.float32),
    )(ug, ig, um, im, w1a, w1b, b1, w2, b2, w3, b3, wog, woh, bo)


def kernel(user_ids, item_ids, ue_gmf, ie_gmf, ue_mlp, ie_mlp,
           W1, b1, W2, b2, W3, b3, Wo, bo):
    ug, ig, um, im = _sc_gather(user_ids, item_ids, ue_gmf, ie_gmf,
                                ue_mlp, ie_mlp)
    w1a, w1b = W1[:D], W1[D:]
    wog = Wo[:D, 0].reshape(1, D)
    woh = Wo[D:, 0].reshape(1, Wo.shape[0] - D)
    return _tc_dense(ug, ig, um, im, w1a, w1b, b1.reshape(1, -1),
                     W2, b2.reshape(1, -1), W3, b3.reshape(1, -1),
                     wog, woh, bo.reshape(1, 1))


# speedup vs baseline: 1.0039x; 1.0039x over previous
import functools

import jax
import jax.numpy as jnp
from jax import lax
from jax.experimental import pallas as pl
from jax.experimental.pallas import tpu as pltpu
from jax.experimental.pallas import tpu_sc as plsc

B = 16384
V = 1000000
D = 64
D2 = 2 * D

_NC = 2
_NS = 16
_NW = _NC * _NS
_BPW = B // _NW
_CHUNK = 256
_NCHUNK = _BPW // _CHUNK
_NBUF = 3
_NSTAGE = 4 * _NCHUNK


def _sc_gather_body(idxu_hbm, idxi_hbm, ug_hbm, ig_hbm, um_hbm, im_hbm,
                    o_ug, o_ig, o_um, o_im,
                    idx_u, idx_i, bufs, gsems, wsems):
    wid = lax.axis_index("s") * _NC + lax.axis_index("c")
    base = wid * _BPW
    pltpu.sync_copy(idxu_hbm.at[pl.ds(base, _BPW)], idx_u)
    pltpu.sync_copy(idxi_hbm.at[pl.ds(base, _BPW)], idx_i)

    srcs = ((ug_hbm, idx_u), (ig_hbm, idx_i), (um_hbm, idx_u), (im_hbm, idx_i))
    outs = (o_ug, o_ig, o_um, o_im)

    def gather(s):
        tbl, idx = srcs[s // _NCHUNK]
        c = s % _NCHUNK
        k = s % _NBUF
        return pltpu.async_copy(tbl.at[idx.at[pl.ds(c * _CHUNK, _CHUNK)]],
                                bufs.at[k], gsems.at[k])

    def writeback(s):
        out = outs[s // _NCHUNK]
        c = s % _NCHUNK
        k = s % _NBUF
        row0 = base + c * _CHUNK
        return pltpu.async_copy(bufs.at[k], out.at[pl.ds(row0, _CHUNK)],
                                wsems.at[k])

    caps_g = {}
    caps_w = {}
    for s in range(min(_NBUF, _NSTAGE)):
        caps_g[s] = gather(s)
    for s in range(_NSTAGE):
        caps_g[s].wait()
        caps_w[s] = writeback(s)
        nxt = s + _NBUF
        if nxt < _NSTAGE:
            caps_w[s].wait()
            caps_g[nxt] = gather(nxt)
    for s in range(max(0, _NSTAGE - _NBUF), _NSTAGE):
        if s in caps_w:
            caps_w[s].wait()


_sc_gather = functools.partial(
    pl.kernel,
    out_type=[jax.ShapeDtypeStruct((B, D2), jnp.float32)] * 4,
    mesh=plsc.VectorSubcoreMesh(core_axis_name="c", subcore_axis_name="s"),
    compiler_params=pltpu.CompilerParams(use_tc_tiling_on_sc=True),
    scratch_types=[
        pltpu.VMEM((_BPW,), jnp.int32),
        pltpu.VMEM((_BPW,), jnp.int32),
        pltpu.VMEM((_NBUF, _CHUNK, D2), jnp.float32),
        pltpu.SemaphoreType.DMA((_NBUF,)),
        pltpu.SemaphoreType.DMA((_NBUF,)),
    ],
)(_sc_gather_body)


_BLK = 2048


def _tc_body(uid, iid, ug2, ig2, um2, im2,
             w1a, w1b, b1, w2, b2, w3, b3, wog, woh, bo, out):
    f32 = jnp.float32
    pu = (uid[...] & 1) == 1
    pi = (iid[...] & 1) == 1
    ug = jnp.where(pu, ug2[:, D:], ug2[:, :D])
    ig = jnp.where(pi, ig2[:, D:], ig2[:, :D])
    um = jnp.where(pu, um2[:, D:], um2[:, :D])
    im = jnp.where(pi, im2[:, D:], im2[:, :D])
    gmf = ug * ig
    h = jnp.dot(um, w1a[...], preferred_element_type=f32)
    h += jnp.dot(im, w1b[...], preferred_element_type=f32)
    h = jnp.maximum(h + b1[...], 0.0)
    h = jnp.maximum(jnp.dot(h, w2[...], preferred_element_type=f32) + b2[...], 0.0)
    h = jnp.maximum(jnp.dot(h, w3[...], preferred_element_type=f32) + b3[...], 0.0)
    logit = jnp.sum(gmf * wog[...], axis=1) + jnp.sum(h * woh[...], axis=1)
    out[...] = jax.nn.sigmoid(logit + bo[0, 0])


def _tc_dense(uid, iid, ug2, ig2, um2, im2,
              w1a, w1b, b1, w2, b2, w3, b3, wog, woh, bo):
    n_blk = B // _BLK
    id_spec = pl.BlockSpec((_BLK, 1), lambda i: (i, 0))
    row_spec = pl.BlockSpec((_BLK, D2), lambda i: (i, 0))
    full = lambda a: pl.BlockSpec(a.shape, lambda i: (0,) * a.ndim)
    return pl.pallas_call(
        _tc_body,
        grid=(n_blk,),
        in_specs=[id_spec, id_spec, row_spec, row_spec, row_spec, row_spec,
                  full(w1a), full(w1b), full(b1), full(w2), full(b2),
                  full(w3), full(b3), full(wog), full(woh), full(bo)],
        out_specs=pl.BlockSpec((_BLK,), lambda i: (i,)),
        out_shape=jax.ShapeDtypeStruct((B,), jnp.float32),
    )(uid, iid, ug2, ig2, um2, im2,
      w1a, w1b, b1, w2, b2, w3, b3, wog, woh, bo)


def kernel(user_ids, item_ids, ue_gmf, ie_gmf, ue_mlp, ie_mlp,
           W1, b1, W2, b2, W3, b3, Wo, bo):
    idx_u = lax.shift_right_logical(user_ids, 1)
    idx_i = lax.shift_right_logical(item_ids, 1)
    t_ug = ue_gmf.reshape(V // 2, D2)
    t_ig = ie_gmf.reshape(V // 2, D2)
    t_um = ue_mlp.reshape(V // 2, D2)
    t_im = ie_mlp.reshape(V // 2, D2)
    ug2, ig2, um2, im2 = _sc_gather(idx_u, idx_i, t_ug, t_ig, t_um, t_im)
    w1a, w1b = W1[:D], W1[D:]
    wog = Wo[:D, 0].reshape(1, D)
    woh = Wo[D:, 0].reshape(1, Wo.shape[0] - D)
    return _tc_dense(user_ids.reshape(B, 1), item_ids.reshape(B, 1),
                     ug2, ig2, um2, im2,
                     w1a, w1b, b1.reshape(1, -1), W2, b2.reshape(1, -1),
                     W3, b3.reshape(1, -1), wog, woh, bo.reshape(1, 1))
